# baseline (speedup 1.0000x reference)
import functools

import jax
import jax.numpy as jnp
from jax import lax
from jax.experimental import pallas as pl
from jax.experimental.pallas import tpu as pltpu
from jax.experimental.pallas import tpu_sc as plsc

N = 10000
C = 128
E = 320000
ALPHA = 0.9
NUM_ITERS = 10

NC = 2
NS = 16
NW = NC * NS
EPW = E // NW
CHUNK = 128
NBUF = 2
NCH = 80
NHALF = 2
SLABH = NCH // NHALF
EPAD = NCH * CHUNK
ROWS_PAD = 10240
TPW = ROWS_PAD // NW
TPS = ROWS_PAD // NS

_MESH = plsc.VectorSubcoreMesh(core_axis_name="c", subcore_axis_name="s")


def _fori(n, body):
    lax.fori_loop(jnp.int32(0), jnp.int32(n), body, 0)


@functools.partial(
    pl.kernel,
    out_type=jax.ShapeDtypeStruct((NC, ROWS_PAD, C), jnp.float32),
    mesh=_MESH,
    scratch_types=[
        pltpu.VMEM((NCH, CHUNK), jnp.int32),
        pltpu.VMEM((NCH, CHUNK), jnp.int32),
        pltpu.VMEM((CHUNK, C), jnp.float32),
        pltpu.VMEM((64, C), jnp.float32),
        pltpu.VMEM_SHARED((ROWS_PAD, C), jnp.float32),
        pltpu.SemaphoreType.DMA,
    ],
)
def _spmm(xs_hbm, rowp_hbm, colp_hbm, out_hbm, rowi, coli, gbuf, zbuf, acc,
          sem):
    cid = lax.axis_index("c")
    sid = lax.axis_index("s")
    w = cid * NS + sid

    pltpu.sync_copy(rowp_hbm.at[w], rowi)
    pltpu.sync_copy(colp_hbm.at[w], coli)

    def _zrow(r, carry):
        for k in range(C // 16):
            zbuf[r, pl.ds(k * 16, 16)] = jnp.zeros((16,), jnp.float32)
        return carry

    _fori(64, _zrow)

    zbase = sid * TPS

    def _zacc(i, carry):
        pltpu.sync_copy(zbuf, acc.at[pl.ds(zbase + i * 64, 64)])
        return carry

    _fori(TPS // 64, _zacc)
    plsc.subcore_barrier()

    def _edge(j, carry):
        pltpu.async_copy(xs_hbm.at[rowi.at[j]], gbuf, sem).wait()
        pltpu.sync_copy(gbuf, acc.at[coli.at[j]], add=True)
        return carry

    _fori(NCH, _edge)
    plsc.subcore_barrier()

    pltpu.sync_copy(acc.at[pl.ds(zbase, TPS)], out_hbm.at[cid, pl.ds(zbase, TPS)])


@functools.partial(
    pl.kernel,
    out_type=jax.ShapeDtypeStruct((ROWS_PAD, C), jnp.float32),
    mesh=_MESH,
    scratch_types=[
        pltpu.VMEM((TPW, C), jnp.float32),
        pltpu.VMEM((TPW, C), jnp.float32),
        pltpu.VMEM((TPW, C), jnp.float32),
        pltpu.VMEM((TPW,), jnp.float32),
    ],
    compiler_params=pltpu.CompilerParams(needs_layout_passes=False),
)
def _combine(part_hbm, scale_hbm, add_hbm, out_hbm, a0, a1, ab, sv):
    w = lax.axis_index("c") * NS + lax.axis_index("s")
    base = w * TPW
    pltpu.sync_copy(part_hbm.at[jnp.int32(0), pl.ds(base, TPW)], a0)
    pltpu.sync_copy(part_hbm.at[jnp.int32(1), pl.ds(base, TPW)], a1)
    pltpu.sync_copy(add_hbm.at[pl.ds(base, TPW)], ab)
    pltpu.sync_copy(scale_hbm.at[pl.ds(base, TPW)], sv)

    def _row(r, carry):
        sc = plsc.load_gather(sv, [jnp.zeros((16,), jnp.int32) + r])
        for k in range(C // 16):
            s = pl.ds(k * 16, 16)
            a0[r, s] = sc * (a0[r, s] + a1[r, s]) + ab[r, s]
        return carry

    _fori(TPW, _row)
    pltpu.sync_copy(a0, out_hbm.at[pl.ds(base, TPW)])


def kernel(edge_index, label, train_idx):
    row = edge_index[0].astype(jnp.int32)
    col = edge_index[1].astype(jnp.int32)
    label = label.astype(jnp.float32)
    ti = train_idx.astype(jnp.int32)

    deg = jnp.zeros((N,), jnp.float32).at[col].add(1.0)
    dis = jnp.where(deg > 0, lax.rsqrt(jnp.maximum(deg, 1.0)), 0.0)
    y = jnp.zeros((N, C), jnp.float32).at[ti].set(label[ti])

    npad = EPAD - EPW
    padc = (jnp.arange(NW, dtype=jnp.int32)[:, None] * 113
            + jnp.arange(npad, dtype=jnp.int32)[None, :] * 37) % N
    rowp = jnp.pad(row.reshape(NW, EPW), ((0, 0), (0, npad)),
                   constant_values=N).reshape(NW, NCH, CHUNK)
    colp = jnp.concatenate([col.reshape(NW, EPW), padc],
                           axis=1).reshape(NW, NCH, CHUNK)

    d2 = dis * dis
    pad1 = (0, ROWS_PAD - N)
    scale_h1 = jnp.pad(d2, pad1)
    scale_h2 = ALPHA * scale_h1
    scale_fin = ALPHA * jnp.pad(dis, pad1)
    add_zero = jnp.zeros((ROWS_PAD, C), jnp.float32)
    yb = jnp.pad((1.0 - ALPHA) * dis[:, None] * y, (pad1, (0, 0)))
    yfin = jnp.pad((1.0 - ALPHA) * y, (pad1, (0, 0)))
    xs = jnp.pad(dis[:, None] * y, (pad1, (0, 0)))

    for i in range(NUM_ITERS):
        part = _spmm(xs, rowp, colp)
        xs = _combine(part, scale_h1, add_zero)
        part = _spmm(xs, rowp, colp)
        if i < NUM_ITERS - 1:
            xs = _combine(part, scale_h2, yb)
        else:
            out = _combine(part, scale_fin, yfin)
    return out[:N]


# speedup vs baseline: 1.4421x; 1.4421x over previous
import functools

import jax
import jax.numpy as jnp
from jax import lax
from jax.experimental import pallas as pl
from jax.experimental.pallas import tpu as pltpu
from jax.experimental.pallas import tpu_sc as plsc

N = 10000
C = 128
E = 320000
ALPHA = 0.9
NUM_ITERS = 10

NC = 2
NS = 16
NW = NC * NS
EPW = E // NW
CHUNK = 128
NBUF = 2
NCH = 79
NHALF = 2
SLABH = NCH // NHALF
EPAD = NCH * CHUNK
ROWS_PAD = 10240
TPW = ROWS_PAD // NW
TPS = ROWS_PAD // NS

_MESH = plsc.VectorSubcoreMesh(core_axis_name="c", subcore_axis_name="s")


def _fori(n, body):
    lax.fori_loop(jnp.int32(0), jnp.int32(n), body, 0)


@functools.partial(
    pl.kernel,
    out_type=jax.ShapeDtypeStruct((NC, ROWS_PAD, C), jnp.float32),
    mesh=_MESH,
    scratch_types=[
        pltpu.VMEM((NCH, CHUNK), jnp.int32),
        pltpu.VMEM((NCH, CHUNK), jnp.int32),
        pltpu.VMEM((CHUNK, C), jnp.float32),
        pltpu.VMEM((64, C), jnp.float32),
        pltpu.VMEM_SHARED((ROWS_PAD, C), jnp.float32),
        pltpu.SemaphoreType.DMA,
    ],
)
def _spmm(xs_hbm, rowp_hbm, colp_hbm, out_hbm, rowi, coli, gbuf, zbuf, acc,
          sem):
    cid = lax.axis_index("c")
    sid = lax.axis_index("s")
    w = cid * NS + sid

    pltpu.sync_copy(rowp_hbm.at[w], rowi)
    pltpu.sync_copy(colp_hbm.at[w], coli)

    def _zrow(r, carry):
        for k in range(C // 16):
            zbuf[r, pl.ds(k * 16, 16)] = jnp.zeros((16,), jnp.float32)
        return carry

    _fori(64, _zrow)

    zbase = sid * TPS

    def _zacc(i, carry):
        pltpu.sync_copy(zbuf, acc.at[pl.ds(zbase + i * 64, 64)])
        return carry

    _fori(TPS // 64, _zacc)
    plsc.subcore_barrier()

    def _edge(j, carry):
        pltpu.async_copy(xs_hbm.at[rowi.at[j]], gbuf, sem).wait()
        pltpu.sync_copy(gbuf, acc.at[coli.at[j]], add=True)
        return carry

    _fori(NCH, _edge)
    plsc.subcore_barrier()

    pltpu.sync_copy(acc.at[pl.ds(zbase, TPS)], out_hbm.at[cid, pl.ds(zbase, TPS)])


@functools.partial(
    pl.kernel,
    out_type=jax.ShapeDtypeStruct((ROWS_PAD, C), jnp.float32),
    mesh=_MESH,
    scratch_types=[
        pltpu.VMEM((TPW, C), jnp.float32),
        pltpu.VMEM((TPW, C), jnp.float32),
        pltpu.VMEM((TPW, C), jnp.float32),
        pltpu.VMEM((TPW,), jnp.float32),
    ],
    compiler_params=pltpu.CompilerParams(needs_layout_passes=False),
)
def _combine(part_hbm, scale_hbm, add_hbm, out_hbm, a0, a1, ab, sv):
    w = lax.axis_index("c") * NS + lax.axis_index("s")
    base = w * TPW
    pltpu.sync_copy(part_hbm.at[jnp.int32(0), pl.ds(base, TPW)], a0)
    pltpu.sync_copy(part_hbm.at[jnp.int32(1), pl.ds(base, TPW)], a1)
    pltpu.sync_copy(add_hbm.at[pl.ds(base, TPW)], ab)
    pltpu.sync_copy(scale_hbm.at[pl.ds(base, TPW)], sv)

    def _row(r, carry):
        sc = plsc.load_gather(sv, [jnp.zeros((16,), jnp.int32) + r])
        for k in range(C // 16):
            s = pl.ds(k * 16, 16)
            a0[r, s] = sc * (a0[r, s] + a1[r, s]) + ab[r, s]
        return carry

    _fori(TPW, _row)
    pltpu.sync_copy(a0, out_hbm.at[pl.ds(base, TPW)])


def kernel(edge_index, label, train_idx):
    row = edge_index[0].astype(jnp.int32)
    col = edge_index[1].astype(jnp.int32)
    label = label.astype(jnp.float32)
    ti = train_idx.astype(jnp.int32)

    deg = jnp.zeros((N,), jnp.float32).at[col].add(1.0)
    dis = jnp.where(deg > 0, lax.rsqrt(jnp.maximum(deg, 1.0)), 0.0)
    y = jnp.zeros((N, C), jnp.float32).at[ti].set(label[ti])

    rowp = jnp.pad(row.reshape(NW, EPW), ((0, 0), (0, EPAD - EPW)),
                   constant_values=0).reshape(NW, NCH, CHUNK)
    colp = jnp.pad(col.reshape(NW, EPW), ((0, 0), (0, EPAD - EPW)),
                   constant_values=N).reshape(NW, NCH, CHUNK)

    d2 = dis * dis
    pad1 = (0, ROWS_PAD - N)
    scale_h1 = jnp.pad(d2, pad1)
    scale_h2 = ALPHA * scale_h1
    scale_fin = ALPHA * jnp.pad(dis, pad1)
    add_zero = jnp.zeros((ROWS_PAD, C), jnp.float32)
    yb = jnp.pad((1.0 - ALPHA) * dis[:, None] * y, (pad1, (0, 0)))
    yfin = jnp.pad((1.0 - ALPHA) * y, (pad1, (0, 0)))
    xs = jnp.pad(dis[:, None] * y, (pad1, (0, 0)))

    for i in range(NUM_ITERS):
        part = _spmm(xs, rowp, colp)
        xs = _combine(part, scale_h1, add_zero)
        part = _spmm(xs, rowp, colp)
        if i < NUM_ITERS - 1:
            xs = _combine(part, scale_h2, yb)
        else:
            out = _combine(part, scale_fin, yfin)
    return out[:N]


# speedup vs baseline: 1.5101x; 1.0472x over previous
import functools

import jax
import jax.numpy as jnp
from jax import lax
from jax.experimental import pallas as pl
from jax.experimental.pallas import tpu as pltpu
from jax.experimental.pallas import tpu_sc as plsc

N = 10000
C = 128
E = 320000
ALPHA = 0.9
NUM_ITERS = 10

NC = 2
NS = 16
NW = NC * NS
EPW = E // NW
CHUNK = 128
NBUF = 2
NCH = 79
NHALF = 2
SLABH = NCH // NHALF
EPAD = NCH * CHUNK
ROWS_PAD = 10240
TPW = ROWS_PAD // NW
TPS = ROWS_PAD // NS

_MESH = plsc.VectorSubcoreMesh(core_axis_name="c", subcore_axis_name="s")


def _fori(n, body):
    lax.fori_loop(jnp.int32(0), jnp.int32(n), body, 0)


@functools.partial(
    pl.kernel,
    out_type=jax.ShapeDtypeStruct((NC, ROWS_PAD, C), jnp.float32),
    mesh=_MESH,
    scratch_types=[
        pltpu.VMEM((NCH, CHUNK), jnp.int32),
        pltpu.VMEM((NCH, CHUNK), jnp.int32),
        pltpu.VMEM((CHUNK, C), jnp.float32),
        pltpu.VMEM((64, C), jnp.float32),
        pltpu.VMEM_SHARED((ROWS_PAD, C), jnp.float32),
        pltpu.SemaphoreType.DMA,
    ],
)
def _spmm(xs_hbm, rowp_hbm, colp_hbm, out_hbm, rowi, coli, gbuf, zbuf, acc,
          sem):
    cid = lax.axis_index("c")
    sid = lax.axis_index("s")
    w = cid * NS + sid

    pltpu.sync_copy(rowp_hbm.at[w], rowi)
    pltpu.sync_copy(colp_hbm.at[w], coli)

    def _zrow(r, carry):
        for k in range(C // 16):
            zbuf[r, pl.ds(k * 16, 16)] = jnp.zeros((16,), jnp.float32)
        return carry

    _fori(64, _zrow)

    zbase = sid * TPS

    def _zacc(i, carry):
        pltpu.sync_copy(zbuf, acc.at[pl.ds(zbase + i * 64, 64)])
        return carry

    _fori(TPS // 64, _zacc)
    plsc.subcore_barrier()

    def _edge(j, carry):
        pltpu.async_copy(xs_hbm.at[rowi.at[j]], gbuf, sem).wait()
        pltpu.sync_copy(gbuf, acc.at[coli.at[j]], add=True)
        return carry

    _fori(NCH, _edge)
    plsc.subcore_barrier()

    pltpu.sync_copy(acc.at[pl.ds(zbase, TPS)], out_hbm.at[cid, pl.ds(zbase, TPS)])


@functools.partial(
    pl.kernel,
    out_type=jax.ShapeDtypeStruct((ROWS_PAD, C), jnp.float32),
    mesh=_MESH,
    scratch_types=[
        pltpu.VMEM((TPW, C), jnp.float32),
        pltpu.VMEM((TPW, C), jnp.float32),
        pltpu.VMEM((TPW, C), jnp.float32),
        pltpu.VMEM((TPW,), jnp.float32),
    ],
    compiler_params=pltpu.CompilerParams(needs_layout_passes=False),
)
def _combine(part_hbm, scale_hbm, add_hbm, out_hbm, a0, a1, ab, sv):
    w = lax.axis_index("c") * NS + lax.axis_index("s")
    base = w * TPW
    pltpu.sync_copy(part_hbm.at[jnp.int32(0), pl.ds(base, TPW)], a0)
    pltpu.sync_copy(part_hbm.at[jnp.int32(1), pl.ds(base, TPW)], a1)
    pltpu.sync_copy(add_hbm.at[pl.ds(base, TPW)], ab)
    pltpu.sync_copy(scale_hbm.at[pl.ds(base, TPW)], sv)

    def _row(r, carry):
        sc = plsc.load_gather(sv, [jnp.zeros((16,), jnp.int32) + r])
        for k in range(C // 16):
            s = pl.ds(k * 16, 16)
            a0[r, s] = sc * (a0[r, s] + a1[r, s]) + ab[r, s]
        return carry

    _fori(TPW, _row)
    pltpu.sync_copy(a0, out_hbm.at[pl.ds(base, TPW)])


NT = 2
SUB = 80
_IOTA16 = None


@functools.partial(
    pl.kernel,
    out_type=(jax.ShapeDtypeStruct((NC, ROWS_PAD, 16), jnp.float32),
              jax.ShapeDtypeStruct((NC, ROWS_PAD, 16), jnp.float32)),
    mesh=_MESH,
    scratch_types=[
        pltpu.VMEM((NCH, CHUNK), jnp.int32),
        pltpu.VMEM((NT, CHUNK), jnp.int32),
        pltpu.VMEM((CHUNK, 16), jnp.float32),
        pltpu.VMEM_SHARED((ROWS_PAD, 16), jnp.float32),
        pltpu.VMEM_SHARED((ROWS_PAD, 16), jnp.float32),
    ],
)
def _prep(colp_hbm, tip_hbm, degp_hbm, maskp_hbm, coli, tsl, ones, dacc, macc):
    cid = lax.axis_index("c")
    sid = lax.axis_index("s")
    w = cid * NS + sid
    pltpu.sync_copy(colp_hbm.at[w], coli)
    pltpu.sync_copy(tip_hbm.at[w], tsl)

    def _zrow(r, carry):
        ones[r, pl.ds(0, 16)] = jnp.zeros((16,), jnp.float32)
        return carry

    _fori(CHUNK, _zrow)
    zbase = sid * TPS

    def _zacc(i, carry):
        pltpu.sync_copy(ones, dacc.at[pl.ds(zbase + i * CHUNK, CHUNK)])
        pltpu.sync_copy(ones, macc.at[pl.ds(zbase + i * CHUNK, CHUNK)])
        return carry

    _fori(TPS // CHUNK, _zacc)

    def _orow(r, carry):
        ones[r, pl.ds(0, 16)] = jnp.zeros((16,), jnp.float32) + jnp.float32(1.0)
        return carry

    _fori(CHUNK, _orow)
    plsc.subcore_barrier()

    def _edge(j, carry):
        pltpu.sync_copy(ones, dacc.at[coli.at[j]], add=True)
        return carry

    _fori(NCH, _edge)
    for t in range(NT):
        pltpu.sync_copy(ones, macc.at[tsl.at[jnp.int32(t)]], add=True)
    plsc.subcore_barrier()

    pltpu.sync_copy(dacc.at[pl.ds(zbase, TPS)],
                    degp_hbm.at[cid, pl.ds(zbase, TPS)])
    pltpu.sync_copy(macc.at[pl.ds(zbase, TPS)],
                    maskp_hbm.at[cid, pl.ds(zbase, TPS)])


@functools.partial(
    pl.kernel,
    out_type=(jax.ShapeDtypeStruct((ROWS_PAD, C), jnp.float32),
              jax.ShapeDtypeStruct((ROWS_PAD, C), jnp.float32),
              jax.ShapeDtypeStruct((ROWS_PAD, C), jnp.float32),
              jax.ShapeDtypeStruct((ROWS_PAD,), jnp.float32),
              jax.ShapeDtypeStruct((ROWS_PAD,), jnp.float32),
              jax.ShapeDtypeStruct((ROWS_PAD,), jnp.float32)),
    mesh=_MESH,
    scratch_types=[
        [pltpu.VMEM((SUB, 16), jnp.float32) for _ in range(4)],
        pltpu.VMEM((SUB, C), jnp.float32),
        [pltpu.VMEM((SUB, C), jnp.float32) for _ in range(3)],
        [pltpu.VMEM((SUB,), jnp.float32) for _ in range(6)],
    ],
    compiler_params=pltpu.CompilerParams(needs_layout_passes=False),
)
def _finish(degp_hbm, maskp_hbm, labelp_hbm,
            xs0_hbm, yb_hbm, yfin_hbm, s1_hbm, s2_hbm, s3_hbm,
            dm, lab, outs, rsc):
    d0, d1, m0, m1 = dm
    o1, o2, o3 = outs
    g1b, g2b, g3b, s1b, s2b, s3b = rsc
    w = lax.axis_index("c") * NS + lax.axis_index("s")
    iota = lax.iota(jnp.int32, 16)
    for ss in range(TPW // SUB):
        sbase = w * TPW + ss * SUB
        pltpu.sync_copy(degp_hbm.at[jnp.int32(0), pl.ds(sbase, SUB)], d0)
        pltpu.sync_copy(degp_hbm.at[jnp.int32(1), pl.ds(sbase, SUB)], d1)
        pltpu.sync_copy(maskp_hbm.at[jnp.int32(0), pl.ds(sbase, SUB)], m0)
        pltpu.sync_copy(maskp_hbm.at[jnp.int32(1), pl.ds(sbase, SUB)], m1)
        pltpu.sync_copy(labelp_hbm.at[pl.ds(sbase, SUB)], lab)

        def _grp(g, carry):
            lvec = g * 16 + iota
            dd = (plsc.load_gather(d0, [lvec, iota])
                  + plsc.load_gather(d1, [lvec, iota]))
            mm = (plsc.load_gather(m0, [lvec, iota])
                  + plsc.load_gather(m1, [lvec, iota]))
            ii = jnp.int32(0x5F3759DF) - lax.shift_right_logical(
                plsc.bitcast(dd, jnp.int32), jnp.int32(1))
            yv = plsc.bitcast(ii, jnp.float32)
            for _ in range(3):
                yv = yv * (jnp.float32(1.5) - jnp.float32(0.5) * dd * yv * yv)
            f0 = jnp.float32(0.0)
            dis = jnp.where(dd > jnp.float32(0.5), yv, f0)
            mf = jnp.where(mm > jnp.float32(0.5), jnp.float32(1.0), f0)
            one_a = jnp.float32(1.0 - ALPHA)
            al = jnp.float32(ALPHA)
            sl = pl.ds(g * 16, 16)
            g1b[sl] = dis * mf
            g2b[sl] = one_a * dis * mf
            g3b[sl] = one_a * mf
            s1b[sl] = dis * dis
            s2b[sl] = al * dis * dis
            s3b[sl] = al * dis
            return carry

        _fori(SUB // 16, _grp)

        def _row(r, carry):
            rv = jnp.zeros((16,), jnp.int32) + r
            b1 = plsc.load_gather(g1b, [rv])
            b2 = plsc.load_gather(g2b, [rv])
            b3 = plsc.load_gather(g3b, [rv])
            for k in range(C // 16):
                sl = pl.ds(k * 16, 16)
                lv = lab[r, sl]
                o1[r, sl] = b1 * lv
                o2[r, sl] = b2 * lv
                o3[r, sl] = b3 * lv
            return carry

        _fori(SUB, _row)
        pltpu.sync_copy(o1, xs0_hbm.at[pl.ds(sbase, SUB)])
        pltpu.sync_copy(o2, yb_hbm.at[pl.ds(sbase, SUB)])
        pltpu.sync_copy(o3, yfin_hbm.at[pl.ds(sbase, SUB)])
        pltpu.sync_copy(s1b, s1_hbm.at[pl.ds(sbase, SUB)])
        pltpu.sync_copy(s2b, s2_hbm.at[pl.ds(sbase, SUB)])
        pltpu.sync_copy(s3b, s3_hbm.at[pl.ds(sbase, SUB)])


def kernel(edge_index, label, train_idx):
    row = edge_index[0].astype(jnp.int32)
    col = edge_index[1].astype(jnp.int32)
    label = label.astype(jnp.float32)
    ti = train_idx.astype(jnp.int32)

    rowp = jnp.pad(row.reshape(NW, EPW), ((0, 0), (0, EPAD - EPW)),
                   constant_values=0).reshape(NW, NCH, CHUNK)
    colp = jnp.pad(col.reshape(NW, EPW), ((0, 0), (0, EPAD - EPW)),
                   constant_values=N).reshape(NW, NCH, CHUNK)
    tpadn = NW * NT * CHUNK - ti.shape[0]
    tpad = N + (jnp.arange(tpadn, dtype=jnp.int32) * 13) % (ROWS_PAD - N)
    tip = jnp.concatenate([ti, tpad]).reshape(NW, NT, CHUNK)
    labelp = jnp.pad(label, ((0, ROWS_PAD - N), (0, 0)))
    add_zero = jnp.zeros((ROWS_PAD, C), jnp.float32)

    degp, maskp = _prep(colp, tip)
    xs, yb, yfin, scale_h1, scale_h2, scale_fin = _finish(degp, maskp, labelp)

    for i in range(NUM_ITERS):
        part = _spmm(xs, rowp, colp)
        xs = _combine(part, scale_h1, add_zero)
        part = _spmm(xs, rowp, colp)
        if i < NUM_ITERS - 1:
            xs = _combine(part, scale_h2, yb)
        else:
            out = _combine(part, scale_fin, yfin)
    return out[:N]


# speedup vs baseline: 1.5160x; 1.0039x over previous
import functools

import jax
import jax.numpy as jnp
from jax import lax
from jax.experimental import pallas as pl
from jax.experimental.pallas import tpu as pltpu
from jax.experimental.pallas import tpu_sc as plsc

N = 10000
C = 128
E = 320000
ALPHA = 0.9
NUM_ITERS = 10

NC = 2
NS = 16
NW = NC * NS
EPW = E // NW
CHUNK = 128
NBUF = 2
NCH = 79
NHALF = 2
SLABH = NCH // NHALF
EPAD = NCH * CHUNK
ROWS_PAD = 10240
TPW = ROWS_PAD // NW
TPS = ROWS_PAD // NS

_MESH = plsc.VectorSubcoreMesh(core_axis_name="c", subcore_axis_name="s")


def _fori(n, body):
    lax.fori_loop(jnp.int32(0), jnp.int32(n), body, 0)


@functools.partial(
    pl.kernel,
    out_type=jax.ShapeDtypeStruct((NC, ROWS_PAD, C), jnp.float32),
    mesh=_MESH,
    scratch_types=[
        pltpu.VMEM((NCH, CHUNK), jnp.int32),
        pltpu.VMEM((NCH, CHUNK), jnp.int32),
        pltpu.VMEM((CHUNK, C), jnp.float32),
        pltpu.VMEM((64, C), jnp.float32),
        pltpu.VMEM_SHARED((ROWS_PAD, C), jnp.float32),
        pltpu.SemaphoreType.DMA,
    ],
)
def _spmm(xs_hbm, rowp_hbm, colp_hbm, out_hbm, rowi, coli, gbuf, zbuf, acc,
          sem):
    cid = lax.axis_index("c")
    sid = lax.axis_index("s")
    w = cid * NS + sid

    pltpu.sync_copy(rowp_hbm.at[w], rowi)
    pltpu.sync_copy(colp_hbm.at[w], coli)

    def _zrow(r, carry):
        for k in range(C // 16):
            zbuf[r, pl.ds(k * 16, 16)] = jnp.zeros((16,), jnp.float32)
        return carry

    _fori(64, _zrow)

    zbase = sid * TPS

    def _zacc(i, carry):
        pltpu.sync_copy(zbuf, acc.at[pl.ds(zbase + i * 64, 64)])
        return carry

    _fori(TPS // 64, _zacc)
    plsc.subcore_barrier()

    def _edge(j, carry):
        pltpu.async_copy(xs_hbm.at[rowi.at[j]], gbuf, sem).wait()
        pltpu.sync_copy(gbuf, acc.at[coli.at[j]], add=True)
        return carry

    _fori(NCH, _edge)
    plsc.subcore_barrier()

    pltpu.sync_copy(acc.at[pl.ds(zbase, TPS)], out_hbm.at[cid, pl.ds(zbase, TPS)])


@functools.partial(
    pl.kernel,
    out_type=jax.ShapeDtypeStruct((ROWS_PAD, C), jnp.float32),
    mesh=_MESH,
    scratch_types=[
        pltpu.VMEM((TPW, C), jnp.float32),
        pltpu.VMEM((TPW, C), jnp.float32),
        pltpu.VMEM((TPW, C), jnp.float32),
        pltpu.VMEM((TPW,), jnp.float32),
    ],
    compiler_params=pltpu.CompilerParams(needs_layout_passes=False),
)
def _combine(part_hbm, scale_hbm, add_hbm, out_hbm, a0, a1, ab, sv):
    w = lax.axis_index("c") * NS + lax.axis_index("s")
    base = w * TPW
    pltpu.sync_copy(part_hbm.at[jnp.int32(0), pl.ds(base, TPW)], a0)
    pltpu.sync_copy(part_hbm.at[jnp.int32(1), pl.ds(base, TPW)], a1)
    pltpu.sync_copy(add_hbm.at[pl.ds(base, TPW)], ab)
    pltpu.sync_copy(scale_hbm.at[pl.ds(base, TPW)], sv)

    def _row(r, carry):
        sc = plsc.load_gather(sv, [jnp.zeros((16,), jnp.int32) + r])
        for k in range(C // 16):
            s = pl.ds(k * 16, 16)
            a0[r, s] = sc * (a0[r, s] + a1[r, s]) + ab[r, s]
        return carry

    _fori(TPW, _row)
    pltpu.sync_copy(a0, out_hbm.at[pl.ds(base, TPW)])


NT = 2
SUB = 80
_IOTA16 = None


@functools.partial(
    pl.kernel,
    out_type=(jax.ShapeDtypeStruct((NC, ROWS_PAD), jnp.float32),
              jax.ShapeDtypeStruct((NC, ROWS_PAD), jnp.float32)),
    mesh=_MESH,
    scratch_types=[
        pltpu.VMEM((NCH, CHUNK), jnp.int32),
        pltpu.VMEM((NT, CHUNK), jnp.int32),
        pltpu.VMEM((CHUNK,), jnp.float32),
        pltpu.VMEM((CHUNK,), jnp.float32),
        pltpu.VMEM_SHARED((ROWS_PAD,), jnp.float32),
        pltpu.VMEM_SHARED((ROWS_PAD,), jnp.float32),
    ],
)
def _prep(colp_hbm, tip_hbm, degp_hbm, maskp_hbm, coli, tsl, zsrc, ones,
          dacc, macc):
    cid = lax.axis_index("c")
    sid = lax.axis_index("s")
    w = cid * NS + sid
    pltpu.sync_copy(colp_hbm.at[w], coli)
    pltpu.sync_copy(tip_hbm.at[w], tsl)

    def _zrow(r, carry):
        sl = pl.ds(r * 16, 16)
        zsrc[sl] = jnp.zeros((16,), jnp.float32)
        ones[sl] = jnp.zeros((16,), jnp.float32) + jnp.float32(1.0)
        return carry

    _fori(CHUNK // 16, _zrow)
    zbase = sid * TPS

    def _zacc(i, carry):
        pltpu.sync_copy(zsrc, dacc.at[pl.ds(zbase + i * CHUNK, CHUNK)])
        pltpu.sync_copy(zsrc, macc.at[pl.ds(zbase + i * CHUNK, CHUNK)])
        return carry

    _fori(TPS // CHUNK, _zacc)
    plsc.subcore_barrier()

    def _edge(j, carry):
        pltpu.sync_copy(ones, dacc.at[coli.at[j]], add=True)
        return carry

    _fori(NCH, _edge)
    for t in range(NT):
        pltpu.sync_copy(ones, macc.at[tsl.at[jnp.int32(t)]], add=True)
    plsc.subcore_barrier()

    pltpu.sync_copy(dacc.at[pl.ds(zbase, TPS)],
                    degp_hbm.at[cid, pl.ds(zbase, TPS)])
    pltpu.sync_copy(macc.at[pl.ds(zbase, TPS)],
                    maskp_hbm.at[cid, pl.ds(zbase, TPS)])


@functools.partial(
    pl.kernel,
    out_type=(jax.ShapeDtypeStruct((ROWS_PAD, C), jnp.float32),
              jax.ShapeDtypeStruct((ROWS_PAD, C), jnp.float32),
              jax.ShapeDtypeStruct((ROWS_PAD, C), jnp.float32),
              jax.ShapeDtypeStruct((ROWS_PAD,), jnp.float32),
              jax.ShapeDtypeStruct((ROWS_PAD,), jnp.float32),
              jax.ShapeDtypeStruct((ROWS_PAD,), jnp.float32)),
    mesh=_MESH,
    scratch_types=[
        [pltpu.VMEM((SUB,), jnp.float32) for _ in range(4)],
        pltpu.VMEM((SUB, C), jnp.float32),
        [pltpu.VMEM((SUB, C), jnp.float32) for _ in range(3)],
        [pltpu.VMEM((SUB,), jnp.float32) for _ in range(6)],
    ],
    compiler_params=pltpu.CompilerParams(needs_layout_passes=False),
)
def _finish(d0_hbm, d1_hbm, m0_hbm, m1_hbm, labelp_hbm,
            xs0_hbm, yb_hbm, yfin_hbm, s1_hbm, s2_hbm, s3_hbm,
            dm, lab, outs, rsc):
    d0, d1, m0, m1 = dm
    o1, o2, o3 = outs
    g1b, g2b, g3b, s1b, s2b, s3b = rsc
    w = lax.axis_index("c") * NS + lax.axis_index("s")
    for ss in range(TPW // SUB):
        sbase = w * TPW + ss * SUB
        pltpu.sync_copy(d0_hbm.at[pl.ds(sbase, SUB)], d0)
        pltpu.sync_copy(d1_hbm.at[pl.ds(sbase, SUB)], d1)
        pltpu.sync_copy(m0_hbm.at[pl.ds(sbase, SUB)], m0)
        pltpu.sync_copy(m1_hbm.at[pl.ds(sbase, SUB)], m1)
        pltpu.sync_copy(labelp_hbm.at[pl.ds(sbase, SUB)], lab)

        def _grp(g, carry):
            sl = pl.ds(g * 16, 16)
            dd = d0[sl] + d1[sl]
            mm = m0[sl] + m1[sl]
            ii = jnp.int32(0x5F3759DF) - lax.shift_right_logical(
                plsc.bitcast(dd, jnp.int32), jnp.int32(1))
            yv = plsc.bitcast(ii, jnp.float32)
            for _ in range(3):
                yv = yv * (jnp.float32(1.5) - jnp.float32(0.5) * dd * yv * yv)
            f0 = jnp.float32(0.0)
            dis = jnp.where(dd > jnp.float32(0.5), yv, f0)
            mf = jnp.where(mm > jnp.float32(0.5), jnp.float32(1.0), f0)
            one_a = jnp.float32(1.0 - ALPHA)
            al = jnp.float32(ALPHA)
            g1b[sl] = dis * mf
            g2b[sl] = one_a * dis * mf
            g3b[sl] = one_a * mf
            s1b[sl] = dis * dis
            s2b[sl] = al * dis * dis
            s3b[sl] = al * dis
            return carry

        _fori(SUB // 16, _grp)

        def _row(r, carry):
            rv = jnp.zeros((16,), jnp.int32) + r
            b1 = plsc.load_gather(g1b, [rv])
            b2 = plsc.load_gather(g2b, [rv])
            b3 = plsc.load_gather(g3b, [rv])
            for k in range(C // 16):
                sl = pl.ds(k * 16, 16)
                lv = lab[r, sl]
                o1[r, sl] = b1 * lv
                o2[r, sl] = b2 * lv
                o3[r, sl] = b3 * lv
            return carry

        _fori(SUB, _row)
        pltpu.sync_copy(o1, xs0_hbm.at[pl.ds(sbase, SUB)])
        pltpu.sync_copy(o2, yb_hbm.at[pl.ds(sbase, SUB)])
        pltpu.sync_copy(o3, yfin_hbm.at[pl.ds(sbase, SUB)])
        pltpu.sync_copy(s1b, s1_hbm.at[pl.ds(sbase, SUB)])
        pltpu.sync_copy(s2b, s2_hbm.at[pl.ds(sbase, SUB)])
        pltpu.sync_copy(s3b, s3_hbm.at[pl.ds(sbase, SUB)])


def kernel(edge_index, label, train_idx):
    row = edge_index[0].astype(jnp.int32)
    col = edge_index[1].astype(jnp.int32)
    label = label.astype(jnp.float32)
    ti = train_idx.astype(jnp.int32)

    rowp = jnp.pad(row.reshape(NW, EPW), ((0, 0), (0, EPAD - EPW)),
                   constant_values=0).reshape(NW, NCH, CHUNK)
    colp = jnp.pad(col.reshape(NW, EPW), ((0, 0), (0, EPAD - EPW)),
                   constant_values=N).reshape(NW, NCH, CHUNK)
    tpadn = NW * NT * CHUNK - ti.shape[0]
    tpad = N + (jnp.arange(tpadn, dtype=jnp.int32) * 13) % (ROWS_PAD - N)
    tip = jnp.concatenate([ti, tpad]).reshape(NW, NT, CHUNK)
    labelp = jnp.pad(label, ((0, ROWS_PAD - N), (0, 0)))
    add_zero = jnp.zeros((ROWS_PAD, C), jnp.float32)

    degp, maskp = _prep(colp, tip)
    xs, yb, yfin, scale_h1, scale_h2, scale_fin = _finish(
        degp[0], degp[1], maskp[0], maskp[1], labelp)

    for i in range(NUM_ITERS):
        part = _spmm(xs, rowp, colp)
        xs = _combine(part, scale_h1, add_zero)
        part = _spmm(xs, rowp, colp)
        if i < NUM_ITERS - 1:
            xs = _combine(part, scale_h2, yb)
        else:
            out = _combine(part, scale_fin, yfin)
    return out[:N]


# speedup vs baseline: 1.5303x; 1.0094x over previous
import functools

import jax
import jax.numpy as jnp
from jax import lax
from jax.experimental import pallas as pl
from jax.experimental.pallas import tpu as pltpu
from jax.experimental.pallas import tpu_sc as plsc

N = 10000
C = 128
E = 320000
ALPHA = 0.9
NUM_ITERS = 10

NC = 2
NS = 16
NW = NC * NS
EPW = E // NW
CHUNK = 128
NBUF = 2
NCH = 79
NHALF = 2
SLABH = NCH // NHALF
EPAD = NCH * CHUNK
ROWS_PAD = 10240
TPW = ROWS_PAD // NW
TPS = ROWS_PAD // NS

_MESH = plsc.VectorSubcoreMesh(core_axis_name="c", subcore_axis_name="s")


def _fori(n, body):
    lax.fori_loop(jnp.int32(0), jnp.int32(n), body, 0)


@functools.partial(
    pl.kernel,
    out_type=jax.ShapeDtypeStruct((NC, ROWS_PAD, C), jnp.float32),
    mesh=_MESH,
    scratch_types=[
        pltpu.VMEM((NCH, CHUNK), jnp.int32),
        pltpu.VMEM((NCH, CHUNK), jnp.int32),
        pltpu.VMEM((CHUNK, C), jnp.float32),
        pltpu.VMEM((64, C), jnp.float32),
        pltpu.VMEM_SHARED((ROWS_PAD, C), jnp.float32),
        pltpu.SemaphoreType.DMA,
    ],
)
def _spmm(xs_hbm, rowp_hbm, colp_hbm, out_hbm, rowi, coli, gbuf, zbuf, acc,
          sem):
    cid = lax.axis_index("c")
    sid = lax.axis_index("s")
    w = cid * NS + sid

    pltpu.sync_copy(rowp_hbm.at[w], rowi)
    pltpu.sync_copy(colp_hbm.at[w], coli)

    def _zrow(r, carry):
        for k in range(C // 16):
            zbuf[r, pl.ds(k * 16, 16)] = jnp.zeros((16,), jnp.float32)
        return carry

    _fori(64, _zrow)

    zbase = sid * TPS

    def _zacc(i, carry):
        pltpu.sync_copy(zbuf, acc.at[pl.ds(zbase + i * 64, 64)])
        return carry

    _fori(TPS // 64, _zacc)
    plsc.subcore_barrier()

    def _edge(j, carry):
        pltpu.async_copy(xs_hbm.at[rowi.at[j]], gbuf, sem).wait()
        pltpu.sync_copy(gbuf, acc.at[coli.at[j]], add=True)
        return carry

    _fori(NCH, _edge)
    plsc.subcore_barrier()

    pltpu.sync_copy(acc.at[pl.ds(zbase, TPS)], out_hbm.at[cid, pl.ds(zbase, TPS)])


def _combine_body(with_add):
    def body(*args):
        if with_add:
            part_hbm, scale_hbm, add_hbm, out_hbm, a0, a1, ab, sv = args
        else:
            part_hbm, scale_hbm, out_hbm, a0, a1, sv = args
        w = lax.axis_index("c") * NS + lax.axis_index("s")
        base = w * TPW
        pltpu.sync_copy(part_hbm.at[jnp.int32(0), pl.ds(base, TPW)], a0)
        pltpu.sync_copy(part_hbm.at[jnp.int32(1), pl.ds(base, TPW)], a1)
        if with_add:
            pltpu.sync_copy(add_hbm.at[pl.ds(base, TPW)], ab)
        pltpu.sync_copy(scale_hbm.at[pl.ds(base, TPW)], sv)

        def _row(r, carry):
            sc = plsc.load_gather(sv, [jnp.zeros((16,), jnp.int32) + r])
            for k in range(C // 16):
                s = pl.ds(k * 16, 16)
                v = sc * (a0[r, s] + a1[r, s])
                a0[r, s] = v + ab[r, s] if with_add else v
            return carry

        _fori(TPW, _row)
        pltpu.sync_copy(a0, out_hbm.at[pl.ds(base, TPW)])

    return body


_combine = pl.kernel(
    _combine_body(True),
    out_type=jax.ShapeDtypeStruct((ROWS_PAD, C), jnp.float32),
    mesh=_MESH,
    scratch_types=[
        pltpu.VMEM((TPW, C), jnp.float32),
        pltpu.VMEM((TPW, C), jnp.float32),
        pltpu.VMEM((TPW, C), jnp.float32),
        pltpu.VMEM((TPW,), jnp.float32),
    ],
    compiler_params=pltpu.CompilerParams(needs_layout_passes=False),
)

_combine_noadd = pl.kernel(
    _combine_body(False),
    out_type=jax.ShapeDtypeStruct((ROWS_PAD, C), jnp.float32),
    mesh=_MESH,
    scratch_types=[
        pltpu.VMEM((TPW, C), jnp.float32),
        pltpu.VMEM((TPW, C), jnp.float32),
        pltpu.VMEM((TPW,), jnp.float32),
    ],
    compiler_params=pltpu.CompilerParams(needs_layout_passes=False),
)


NT = 2
SUB = 80
_IOTA16 = None


@functools.partial(
    pl.kernel,
    out_type=(jax.ShapeDtypeStruct((NC, ROWS_PAD), jnp.float32),
              jax.ShapeDtypeStruct((NC, ROWS_PAD), jnp.float32)),
    mesh=_MESH,
    scratch_types=[
        pltpu.VMEM((NCH, CHUNK), jnp.int32),
        pltpu.VMEM((NT, CHUNK), jnp.int32),
        pltpu.VMEM((CHUNK,), jnp.float32),
        pltpu.VMEM((CHUNK,), jnp.float32),
        pltpu.VMEM_SHARED((ROWS_PAD,), jnp.float32),
        pltpu.VMEM_SHARED((ROWS_PAD,), jnp.float32),
    ],
)
def _prep(colp_hbm, tip_hbm, degp_hbm, maskp_hbm, coli, tsl, zsrc, ones,
          dacc, macc):
    cid = lax.axis_index("c")
    sid = lax.axis_index("s")
    w = cid * NS + sid
    pltpu.sync_copy(colp_hbm.at[w], coli)
    pltpu.sync_copy(tip_hbm.at[w], tsl)

    def _zrow(r, carry):
        sl = pl.ds(r * 16, 16)
        zsrc[sl] = jnp.zeros((16,), jnp.float32)
        ones[sl] = jnp.zeros((16,), jnp.float32) + jnp.float32(1.0)
        return carry

    _fori(CHUNK // 16, _zrow)
    zbase = sid * TPS

    def _zacc(i, carry):
        pltpu.sync_copy(zsrc, dacc.at[pl.ds(zbase + i * CHUNK, CHUNK)])
        pltpu.sync_copy(zsrc, macc.at[pl.ds(zbase + i * CHUNK, CHUNK)])
        return carry

    _fori(TPS // CHUNK, _zacc)
    plsc.subcore_barrier()

    def _edge(j, carry):
        pltpu.sync_copy(ones, dacc.at[coli.at[j]], add=True)
        return carry

    _fori(NCH, _edge)
    for t in range(NT):
        pltpu.sync_copy(ones, macc.at[tsl.at[jnp.int32(t)]], add=True)
    plsc.subcore_barrier()

    pltpu.sync_copy(dacc.at[pl.ds(zbase, TPS)],
                    degp_hbm.at[cid, pl.ds(zbase, TPS)])
    pltpu.sync_copy(macc.at[pl.ds(zbase, TPS)],
                    maskp_hbm.at[cid, pl.ds(zbase, TPS)])


@functools.partial(
    pl.kernel,
    out_type=(jax.ShapeDtypeStruct((ROWS_PAD, C), jnp.float32),
              jax.ShapeDtypeStruct((ROWS_PAD, C), jnp.float32),
              jax.ShapeDtypeStruct((ROWS_PAD, C), jnp.float32),
              jax.ShapeDtypeStruct((ROWS_PAD,), jnp.float32),
              jax.ShapeDtypeStruct((ROWS_PAD,), jnp.float32),
              jax.ShapeDtypeStruct((ROWS_PAD,), jnp.float32)),
    mesh=_MESH,
    scratch_types=[
        [pltpu.VMEM((SUB,), jnp.float32) for _ in range(4)],
        pltpu.VMEM((SUB, C), jnp.float32),
        [pltpu.VMEM((SUB, C), jnp.float32) for _ in range(3)],
        [pltpu.VMEM((SUB,), jnp.float32) for _ in range(6)],
    ],
    compiler_params=pltpu.CompilerParams(needs_layout_passes=False),
)
def _finish(d0_hbm, d1_hbm, m0_hbm, m1_hbm, labelp_hbm,
            xs0_hbm, yb_hbm, yfin_hbm, s1_hbm, s2_hbm, s3_hbm,
            dm, lab, outs, rsc):
    d0, d1, m0, m1 = dm
    o1, o2, o3 = outs
    g1b, g2b, g3b, s1b, s2b, s3b = rsc
    w = lax.axis_index("c") * NS + lax.axis_index("s")
    for ss in range(TPW // SUB):
        sbase = w * TPW + ss * SUB
        pltpu.sync_copy(d0_hbm.at[pl.ds(sbase, SUB)], d0)
        pltpu.sync_copy(d1_hbm.at[pl.ds(sbase, SUB)], d1)
        pltpu.sync_copy(m0_hbm.at[pl.ds(sbase, SUB)], m0)
        pltpu.sync_copy(m1_hbm.at[pl.ds(sbase, SUB)], m1)
        pltpu.sync_copy(labelp_hbm.at[pl.ds(sbase, SUB)], lab)

        def _grp(g, carry):
            sl = pl.ds(g * 16, 16)
            dd = d0[sl] + d1[sl]
            mm = m0[sl] + m1[sl]
            ii = jnp.int32(0x5F3759DF) - lax.shift_right_logical(
                plsc.bitcast(dd, jnp.int32), jnp.int32(1))
            yv = plsc.bitcast(ii, jnp.float32)
            for _ in range(3):
                yv = yv * (jnp.float32(1.5) - jnp.float32(0.5) * dd * yv * yv)
            f0 = jnp.float32(0.0)
            dis = jnp.where(dd > jnp.float32(0.5), yv, f0)
            mf = jnp.where(mm > jnp.float32(0.5), jnp.float32(1.0), f0)
            one_a = jnp.float32(1.0 - ALPHA)
            al = jnp.float32(ALPHA)
            g1b[sl] = dis * mf
            g2b[sl] = one_a * dis * mf
            g3b[sl] = one_a * mf
            s1b[sl] = dis * dis
            s2b[sl] = al * dis * dis
            s3b[sl] = al * dis
            return carry

        _fori(SUB // 16, _grp)

        def _row(r, carry):
            rv = jnp.zeros((16,), jnp.int32) + r
            b1 = plsc.load_gather(g1b, [rv])
            b2 = plsc.load_gather(g2b, [rv])
            b3 = plsc.load_gather(g3b, [rv])
            for k in range(C // 16):
                sl = pl.ds(k * 16, 16)
                lv = lab[r, sl]
                o1[r, sl] = b1 * lv
                o2[r, sl] = b2 * lv
                o3[r, sl] = b3 * lv
            return carry

        _fori(SUB, _row)
        pltpu.sync_copy(o1, xs0_hbm.at[pl.ds(sbase, SUB)])
        pltpu.sync_copy(o2, yb_hbm.at[pl.ds(sbase, SUB)])
        pltpu.sync_copy(o3, yfin_hbm.at[pl.ds(sbase, SUB)])
        pltpu.sync_copy(s1b, s1_hbm.at[pl.ds(sbase, SUB)])
        pltpu.sync_copy(s2b, s2_hbm.at[pl.ds(sbase, SUB)])
        pltpu.sync_copy(s3b, s3_hbm.at[pl.ds(sbase, SUB)])


def kernel(edge_index, label, train_idx):
    row = edge_index[0].astype(jnp.int32)
    col = edge_index[1].astype(jnp.int32)
    label = label.astype(jnp.float32)
    ti = train_idx.astype(jnp.int32)

    rowp = jnp.pad(row.reshape(NW, EPW), ((0, 0), (0, EPAD - EPW)),
                   constant_values=0).reshape(NW, NCH, CHUNK)
    colp = jnp.pad(col.reshape(NW, EPW), ((0, 0), (0, EPAD - EPW)),
                   constant_values=N).reshape(NW, NCH, CHUNK)
    tpadn = NW * NT * CHUNK - ti.shape[0]
    tpad = N + (jnp.arange(tpadn, dtype=jnp.int32) * 13) % (ROWS_PAD - N)
    tip = jnp.concatenate([ti, tpad]).reshape(NW, NT, CHUNK)
    labelp = jnp.pad(label, ((0, ROWS_PAD - N), (0, 0)))

    degp, maskp = _prep(colp, tip)
    xs, yb, yfin, scale_h1, scale_h2, scale_fin = _finish(
        degp[0], degp[1], maskp[0], maskp[1], labelp)

    for i in range(NUM_ITERS):
        part = _spmm(xs, rowp, colp)
        xs = _combine_noadd(part, scale_h1)
        part = _spmm(xs, rowp, colp)
        if i < NUM_ITERS - 1:
            xs = _combine(part, scale_h2, yb)
        else:
            out = _combine(part, scale_fin, yfin)
    return out[:N]


# speedup vs baseline: 1.5317x; 1.0009x over previous
import functools

import jax
import jax.numpy as jnp
from jax import lax
from jax.experimental import pallas as pl
from jax.experimental.pallas import tpu as pltpu
from jax.experimental.pallas import tpu_sc as plsc

N = 10000
C = 128
E = 320000
ALPHA = 0.9
NUM_ITERS = 10

NC = 2
NS = 16
NW = NC * NS
EPW = E // NW
CHUNK = 128
NCH = 79
EPAD = NCH * CHUNK
ROWS_PAD = 10240
TPW = ROWS_PAD // NW
TPS = ROWS_PAD // NS

_MESH = plsc.VectorSubcoreMesh(core_axis_name="c", subcore_axis_name="s")


def _fori(n, body):
    lax.fori_loop(jnp.int32(0), jnp.int32(n), body, 0)


@functools.partial(
    pl.kernel,
    out_type=jax.ShapeDtypeStruct((NC, ROWS_PAD, C), jnp.float32),
    mesh=_MESH,
    scratch_types=[
        pltpu.VMEM((NCH, CHUNK), jnp.int32),
        pltpu.VMEM((NCH, CHUNK), jnp.int32),
        pltpu.VMEM((CHUNK, C), jnp.float32),
        pltpu.VMEM((64, C), jnp.float32),
        pltpu.VMEM_SHARED((ROWS_PAD, C), jnp.float32),
        pltpu.SemaphoreType.DMA,
    ],
)
def _spmm(xs_hbm, rowp_hbm, colp_hbm, out_hbm, rowi, coli, gbuf, zbuf, acc,
          sem):
    cid = lax.axis_index("c")
    sid = lax.axis_index("s")
    w = cid * NS + sid

    pltpu.sync_copy(rowp_hbm.at[w], rowi)
    pltpu.sync_copy(colp_hbm.at[w], coli)

    def _zrow(r, carry):
        for k in range(C // 16):
            zbuf[r, pl.ds(k * 16, 16)] = jnp.zeros((16,), jnp.float32)
        return carry

    _fori(64, _zrow)

    zbase = sid * TPS

    def _zacc(i, carry):
        pltpu.sync_copy(zbuf, acc.at[pl.ds(zbase + i * 64, 64)])
        return carry

    _fori(TPS // 64, _zacc)
    plsc.subcore_barrier()

    def _edge(j, carry):
        pltpu.async_copy(xs_hbm.at[rowi.at[j]], gbuf, sem).wait()
        pltpu.sync_copy(gbuf, acc.at[coli.at[j]], add=True)
        return carry

    _fori(NCH, _edge)
    plsc.subcore_barrier()

    pltpu.sync_copy(acc.at[pl.ds(zbase, TPS)], out_hbm.at[cid, pl.ds(zbase, TPS)])


def _combine_body(with_add):
    def body(*args):
        if with_add:
            part_hbm, scale_hbm, add_hbm, out_hbm, a0, a1, ab, sv = args
        else:
            part_hbm, scale_hbm, out_hbm, a0, a1, sv = args
        w = lax.axis_index("c") * NS + lax.axis_index("s")
        base = w * TPW
        pltpu.sync_copy(part_hbm.at[jnp.int32(0), pl.ds(base, TPW)], a0)
        pltpu.sync_copy(part_hbm.at[jnp.int32(1), pl.ds(base, TPW)], a1)
        if with_add:
            pltpu.sync_copy(add_hbm.at[pl.ds(base, TPW)], ab)
        pltpu.sync_copy(scale_hbm.at[pl.ds(base, TPW)], sv)

        def _row(r, carry):
            sc = plsc.load_gather(sv, [jnp.zeros((16,), jnp.int32) + r])
            for k in range(C // 16):
                s = pl.ds(k * 16, 16)
                v = sc * (a0[r, s] + a1[r, s])
                a0[r, s] = v + ab[r, s] if with_add else v
            return carry

        _fori(TPW, _row)
        pltpu.sync_copy(a0, out_hbm.at[pl.ds(base, TPW)])

    return body


_combine = pl.kernel(
    _combine_body(True),
    out_type=jax.ShapeDtypeStruct((ROWS_PAD, C), jnp.float32),
    mesh=_MESH,
    scratch_types=[
        pltpu.VMEM((TPW, C), jnp.float32),
        pltpu.VMEM((TPW, C), jnp.float32),
        pltpu.VMEM((TPW, C), jnp.float32),
        pltpu.VMEM((TPW,), jnp.float32),
    ],
    compiler_params=pltpu.CompilerParams(needs_layout_passes=False),
)

_combine_noadd = pl.kernel(
    _combine_body(False),
    out_type=jax.ShapeDtypeStruct((ROWS_PAD, C), jnp.float32),
    mesh=_MESH,
    scratch_types=[
        pltpu.VMEM((TPW, C), jnp.float32),
        pltpu.VMEM((TPW, C), jnp.float32),
        pltpu.VMEM((TPW,), jnp.float32),
    ],
    compiler_params=pltpu.CompilerParams(needs_layout_passes=False),
)


NT = 2
SUB = 80


@functools.partial(
    pl.kernel,
    out_type=(jax.ShapeDtypeStruct((NC, ROWS_PAD), jnp.float32),
              jax.ShapeDtypeStruct((NC, ROWS_PAD), jnp.float32)),
    mesh=_MESH,
    scratch_types=[
        pltpu.VMEM((NCH, CHUNK), jnp.int32),
        pltpu.VMEM((NT, CHUNK), jnp.int32),
        pltpu.VMEM((CHUNK,), jnp.float32),
        pltpu.VMEM((CHUNK,), jnp.float32),
        pltpu.VMEM_SHARED((ROWS_PAD,), jnp.float32),
        pltpu.VMEM_SHARED((ROWS_PAD,), jnp.float32),
    ],
)
def _prep(colp_hbm, tip_hbm, degp_hbm, maskp_hbm, coli, tsl, zsrc, ones,
          dacc, macc):
    cid = lax.axis_index("c")
    sid = lax.axis_index("s")
    w = cid * NS + sid
    pltpu.sync_copy(colp_hbm.at[w], coli)
    pltpu.sync_copy(tip_hbm.at[w], tsl)

    def _zrow(r, carry):
        sl = pl.ds(r * 16, 16)
        zsrc[sl] = jnp.zeros((16,), jnp.float32)
        ones[sl] = jnp.zeros((16,), jnp.float32) + jnp.float32(1.0)
        return carry

    _fori(CHUNK // 16, _zrow)
    zbase = sid * TPS

    def _zacc(i, carry):
        pltpu.sync_copy(zsrc, dacc.at[pl.ds(zbase + i * CHUNK, CHUNK)])
        pltpu.sync_copy(zsrc, macc.at[pl.ds(zbase + i * CHUNK, CHUNK)])
        return carry

    _fori(TPS // CHUNK, _zacc)
    plsc.subcore_barrier()

    def _edge(j, carry):
        pltpu.sync_copy(ones, dacc.at[coli.at[j]], add=True)
        return carry

    _fori(NCH, _edge)
    for t in range(NT):
        pltpu.sync_copy(ones, macc.at[tsl.at[jnp.int32(t)]], add=True)
    plsc.subcore_barrier()

    pltpu.sync_copy(dacc.at[pl.ds(zbase, TPS)],
                    degp_hbm.at[cid, pl.ds(zbase, TPS)])
    pltpu.sync_copy(macc.at[pl.ds(zbase, TPS)],
                    maskp_hbm.at[cid, pl.ds(zbase, TPS)])


@functools.partial(
    pl.kernel,
    out_type=(jax.ShapeDtypeStruct((ROWS_PAD, C), jnp.float32),
              jax.ShapeDtypeStruct((ROWS_PAD, C), jnp.float32),
              jax.ShapeDtypeStruct((ROWS_PAD, C), jnp.float32),
              jax.ShapeDtypeStruct((ROWS_PAD,), jnp.float32),
              jax.ShapeDtypeStruct((ROWS_PAD,), jnp.float32),
              jax.ShapeDtypeStruct((ROWS_PAD,), jnp.float32)),
    mesh=_MESH,
    scratch_types=[
        [pltpu.VMEM((SUB,), jnp.float32) for _ in range(4)],
        pltpu.VMEM((SUB, C), jnp.float32),
        [pltpu.VMEM((SUB, C), jnp.float32) for _ in range(3)],
        [pltpu.VMEM((SUB,), jnp.float32) for _ in range(6)],
    ],
    compiler_params=pltpu.CompilerParams(needs_layout_passes=False),
)
def _finish(d0_hbm, d1_hbm, m0_hbm, m1_hbm, labelp_hbm,
            xs0_hbm, yb_hbm, yfin_hbm, s1_hbm, s2_hbm, s3_hbm,
            dm, lab, outs, rsc):
    d0, d1, m0, m1 = dm
    o1, o2, o3 = outs
    g1b, g2b, g3b, s1b, s2b, s3b = rsc
    w = lax.axis_index("c") * NS + lax.axis_index("s")
    for ss in range(TPW // SUB):
        sbase = w * TPW + ss * SUB
        pltpu.sync_copy(d0_hbm.at[pl.ds(sbase, SUB)], d0)
        pltpu.sync_copy(d1_hbm.at[pl.ds(sbase, SUB)], d1)
        pltpu.sync_copy(m0_hbm.at[pl.ds(sbase, SUB)], m0)
        pltpu.sync_copy(m1_hbm.at[pl.ds(sbase, SUB)], m1)
        pltpu.sync_copy(labelp_hbm.at[pl.ds(sbase, SUB)], lab)

        def _grp(g, carry):
            sl = pl.ds(g * 16, 16)
            dd = d0[sl] + d1[sl]
            mm = m0[sl] + m1[sl]
            ii = jnp.int32(0x5F3759DF) - lax.shift_right_logical(
                plsc.bitcast(dd, jnp.int32), jnp.int32(1))
            yv = plsc.bitcast(ii, jnp.float32)
            for _ in range(3):
                yv = yv * (jnp.float32(1.5) - jnp.float32(0.5) * dd * yv * yv)
            f0 = jnp.float32(0.0)
            dis = jnp.where(dd > jnp.float32(0.5), yv, f0)
            mf = jnp.where(mm > jnp.float32(0.5), jnp.float32(1.0), f0)
            one_a = jnp.float32(1.0 - ALPHA)
            al = jnp.float32(ALPHA)
            g1b[sl] = dis * mf
            g2b[sl] = one_a * dis * mf
            g3b[sl] = one_a * mf
            s1b[sl] = dis * dis
            s2b[sl] = al * dis * dis
            s3b[sl] = al * dis
            return carry

        _fori(SUB // 16, _grp)

        def _row(r, carry):
            rv = jnp.zeros((16,), jnp.int32) + r
            b1 = plsc.load_gather(g1b, [rv])
            b2 = plsc.load_gather(g2b, [rv])
            b3 = plsc.load_gather(g3b, [rv])
            for k in range(C // 16):
                sl = pl.ds(k * 16, 16)
                lv = lab[r, sl]
                o1[r, sl] = b1 * lv
                o2[r, sl] = b2 * lv
                o3[r, sl] = b3 * lv
            return carry

        _fori(SUB, _row)
        pltpu.sync_copy(o1, xs0_hbm.at[pl.ds(sbase, SUB)])
        pltpu.sync_copy(o2, yb_hbm.at[pl.ds(sbase, SUB)])
        pltpu.sync_copy(o3, yfin_hbm.at[pl.ds(sbase, SUB)])
        pltpu.sync_copy(s1b, s1_hbm.at[pl.ds(sbase, SUB)])
        pltpu.sync_copy(s2b, s2_hbm.at[pl.ds(sbase, SUB)])
        pltpu.sync_copy(s3b, s3_hbm.at[pl.ds(sbase, SUB)])


def kernel(edge_index, label, train_idx):
    row = edge_index[0].astype(jnp.int32)
    col = edge_index[1].astype(jnp.int32)
    label = label.astype(jnp.float32)
    ti = train_idx.astype(jnp.int32)

    rowp = jnp.pad(row.reshape(NW, EPW), ((0, 0), (0, EPAD - EPW)),
                   constant_values=0).reshape(NW, NCH, CHUNK)
    colp = jnp.pad(col.reshape(NW, EPW), ((0, 0), (0, EPAD - EPW)),
                   constant_values=N).reshape(NW, NCH, CHUNK)
    tpadn = NW * NT * CHUNK - ti.shape[0]
    tpad = N + (jnp.arange(tpadn, dtype=jnp.int32) * 13) % (ROWS_PAD - N)
    tip = jnp.concatenate([ti, tpad]).reshape(NW, NT, CHUNK)
    labelp = jnp.pad(label, ((0, ROWS_PAD - N), (0, 0)))

    degp, maskp = _prep(colp, tip)
    xs, yb, yfin, scale_h1, scale_h2, scale_fin = _finish(
        degp[0], degp[1], maskp[0], maskp[1], labelp)

    for i in range(NUM_ITERS):
        part = _spmm(xs, rowp, colp)
        xs = _combine_noadd(part, scale_h1)
        part = _spmm(xs, rowp, colp)
        if i < NUM_ITERS - 1:
            xs = _combine(part, scale_h2, yb)
        else:
            out = _combine(part, scale_fin, yfin)
    return out[:N]
